# Initial kernel scaffold; baseline (speedup 1.0000x reference)
import jax
import jax.numpy as jnp
from jax.experimental import pallas as pl


def kernel(x, codebook):
    raise NotImplementedError("write your pallas kernel here")


# speedup vs baseline: 1.6376x; 1.6376x over previous
import functools

import jax
import jax.numpy as jnp
from jax import lax
from jax.experimental import pallas as pl
from jax.experimental.pallas import tpu as pltpu
from jax.experimental.pallas import tpu_sc as plsc

_B, _D, _L = 8, 64, 1024
_K = 8192
_N = _B * _L
_COMMIT = 0.25

_R = _N + _K
_RT = 4096
_NT = 2048
_KT = 2048
_GRID_N = _N // _NT
_GRID_K = _K // _KT


def _normalize_body(x_ref, out_ref, nrm_ref):
    t = x_ref[...]
    ss = jnp.sum(t * t, axis=1, keepdims=True)
    nrm = jnp.sqrt(ss)
    den = jnp.maximum(nrm, 1e-12)
    out_ref[...] = jnp.concatenate(
        [t / den, jnp.zeros((t.shape[0], 128 - _D), jnp.float32)], axis=1)
    nrm_ref[...] = nrm


def _normalize(stacked, interpret=False):
    return pl.pallas_call(
        _normalize_body,
        grid=(_R // _RT,),
        in_specs=[pl.BlockSpec((_RT, _D), lambda i: (i, 0))],
        out_specs=[
            pl.BlockSpec((_RT, 128), lambda i: (i, 0)),
            pl.BlockSpec((_RT, 1), lambda i: (i, 0)),
        ],
        out_shape=[
            jax.ShapeDtypeStruct((_R, 128), jnp.float32),
            jax.ShapeDtypeStruct((_R, 1), jnp.float32),
        ],
        interpret=interpret,
    )(stacked)


def _argmax_body(fn_ref, cn_ref, nrm_ref, idx_ref, commit_ref, m_sc, kb_sc):
    n = pl.program_id(0)
    k = pl.program_id(1)

    @pl.when(k == 0)
    def _():
        m_sc[...] = jnp.full_like(m_sc[...], -jnp.inf)
        kb_sc[...] = jnp.zeros_like(kb_sc[...])

    sim = lax.dot_general(
        fn_ref[...], cn_ref[...],
        (((1,), (1,)), ((), ())),
        preferred_element_type=jnp.float32,
    )

    m = m_sc[...]
    kb = kb_sc[...]
    for g in range(_KT // 128):
        s = sim[:, g * 128:(g + 1) * 128]
        kbase = k * _KT + g * 128
        upd = s > m
        m = jnp.where(upd, s, m)
        kb = jnp.where(upd, jnp.int32(kbase), kb)
    m_sc[...] = m
    kb_sc[...] = kb

    @pl.when(k == _GRID_K - 1)
    def _():
        mm = m_sc[...]
        row_max = jnp.max(mm, axis=1, keepdims=True)
        kglob = kb_sc[...] + lax.broadcasted_iota(jnp.int32, mm.shape, 1)
        cand = jnp.where(mm == row_max, kglob, jnp.int32(1 << 30))
        idx_ref[...] = jnp.min(cand, axis=1, keepdims=True) + jnp.int32(_N)

        nrm = nrm_ref[...]
        part = jnp.sum(nrm * nrm - 2.0 * nrm * row_max + 1.0,
                       keepdims=True)[:, :1]

        @pl.when(n == 0)
        def _():
            commit_ref[...] = jnp.zeros_like(commit_ref[...])
        commit_ref[...] += part * (_COMMIT / (_N * _D))


def _argmax(normed, norms, interpret=False):
    return pl.pallas_call(
        _argmax_body,
        grid=(_GRID_N, _GRID_K),
        in_specs=[
            pl.BlockSpec((_NT, 128), lambda n, k: (n, 0)),
            pl.BlockSpec((_KT, 128), lambda n, k: (_N // _KT + k, 0)),
            pl.BlockSpec((_NT, 1), lambda n, k: (n, 0)),
        ],
        out_specs=[
            pl.BlockSpec((_NT, 1), lambda n, k: (n, 0)),
            pl.BlockSpec((1, 1), lambda n, k: (0, 0)),
        ],
        out_shape=[
            jax.ShapeDtypeStruct((_N, 1), jnp.int32),
            jax.ShapeDtypeStruct((1, 1), jnp.float32),
        ],
        scratch_shapes=[
            pltpu.VMEM((_NT, 128), jnp.float32),
            pltpu.VMEM((_NT, 128), jnp.int32),
        ],
        interpret=interpret,
    )(normed, normed, norms)


def _gather(normed, idx):
    info = plsc.get_sparse_core_info()
    nc, ns = info.num_cores, info.num_subcores
    nw = nc * ns
    b_per_w = _N // nw
    mesh = plsc.VectorSubcoreMesh(core_axis_name="c", subcore_axis_name="s")

    @functools.partial(
        pl.kernel,
        out_type=jax.ShapeDtypeStruct((_N, 128), jnp.float32),
        mesh=mesh,
        scratch_types=[
            pltpu.VMEM((b_per_w,), jnp.int32),
            pltpu.VMEM((b_per_w, 128), jnp.float32),
            pltpu.SemaphoreType.DMA,
        ],
    )
    def gather_k(table_hbm, idx_hbm, out_hbm, idx_v, rows_v, sem):
        wid = lax.axis_index("s") * nc + lax.axis_index("c")
        base = wid * b_per_w
        pltpu.sync_copy(idx_hbm.at[pl.ds(base, b_per_w)], idx_v)
        pltpu.async_copy(table_hbm.at[idx_v], rows_v, sem).wait()
        pltpu.sync_copy(rows_v, out_hbm.at[pl.ds(base, b_per_w)])

    return gather_k(normed, idx)


def kernel(x, codebook):
    flat = jnp.transpose(x, (0, 2, 1)).reshape(_N, _D)
    stacked = jnp.concatenate([flat, codebook], axis=0)
    normed, norms = _normalize(stacked)
    idx, commit = _argmax(normed, norms)
    q_flat = _gather(normed, idx.reshape(_N))
    q = jnp.transpose(q_flat.reshape(_B, _L, 128)[:, :, :_D], (0, 2, 1))
    return q, commit.reshape(())


# speedup vs baseline: 1.9410x; 1.1853x over previous
import functools

import jax
import jax.numpy as jnp
from jax import lax
from jax.experimental import pallas as pl
from jax.experimental.pallas import tpu as pltpu
from jax.experimental.pallas import tpu_sc as plsc

_B, _D, _L = 8, 64, 1024
_K = 8192
_N = _B * _L
_COMMIT = 0.25

_NT = 2048
_BT = _NT // _L
_CK = 256
_GRID_N = _N // _NT


def _vq_body(x_ref, cb_ref, idx_ref, commit_ref, cn_out_ref, cn_sc):
    n = pl.program_id(0)

    @pl.when(n == 0)
    def _():
        t = cb_ref[...]
        ss = jnp.sum(t * t, axis=1, keepdims=True)
        den = jnp.maximum(jnp.sqrt(ss), 1e-12)
        cn = jnp.concatenate(
            [t / den, jnp.zeros((_K, 128 - _D), jnp.float32)], axis=1)
        cn_sc[...] = cn
        cn_out_ref[...] = cn

    xt = jnp.transpose(x_ref[...], (0, 2, 1)).reshape(_NT, _D)
    ss = jnp.sum(xt * xt, axis=1, keepdims=True)
    nrm = jnp.sqrt(ss)
    den = jnp.maximum(nrm, 1e-12)
    fn = jnp.concatenate(
        [xt / den, jnp.zeros((_NT, 128 - _D), jnp.float32)], axis=1)

    m = jnp.full((_NT, 128), -jnp.inf, jnp.float32)
    aux = jnp.full((_NT, 128), -jnp.inf, jnp.float32)
    kb = jnp.zeros((_NT, 128), jnp.int32)
    for c in range(_K // _CK):
        sim = lax.dot_general(
            fn, cn_sc[c * _CK:(c + 1) * _CK, :],
            (((1,), (1,)), ((), ())),
            preferred_element_type=jnp.float32,
        )
        s_a = sim[:, 0:128]
        s_b = sim[:, 128:256]
        mm = jnp.maximum(s_a, s_b)
        upd = mm > m
        m = jnp.where(upd, mm, m)
        kb = jnp.where(upd, jnp.int32(c), kb)
        aux = jnp.where(upd, s_a, aux)

    row_max = jnp.max(m, axis=1, keepdims=True)
    kglob = (kb * _CK
             + jnp.where(aux == m, jnp.int32(0), jnp.int32(128))
             + lax.broadcasted_iota(jnp.int32, m.shape, 1))
    cand = jnp.where(m == row_max, kglob, jnp.int32(1 << 30))
    idx_ref[...] = jnp.min(cand, axis=1, keepdims=True)

    part = jnp.sum(nrm * nrm - 2.0 * nrm * row_max + 1.0,
                   keepdims=True)[:, :1]

    @pl.when(n == 0)
    def _():
        commit_ref[...] = jnp.zeros_like(commit_ref[...])
    commit_ref[...] += part * (_COMMIT / (_N * _D))


def _vq_argmax(x, codebook, interpret=False):
    return pl.pallas_call(
        _vq_body,
        grid=(_GRID_N,),
        in_specs=[
            pl.BlockSpec((_BT, _D, _L), lambda n: (n, 0, 0)),
            pl.BlockSpec((_K, _D), lambda n: (0, 0)),
        ],
        out_specs=[
            pl.BlockSpec((_NT, 1), lambda n: (n, 0)),
            pl.BlockSpec((1, 1), lambda n: (0, 0)),
            pl.BlockSpec((_K, 128), lambda n: (0, 0)),
        ],
        out_shape=[
            jax.ShapeDtypeStruct((_N, 1), jnp.int32),
            jax.ShapeDtypeStruct((1, 1), jnp.float32),
            jax.ShapeDtypeStruct((_K, 128), jnp.float32),
        ],
        scratch_shapes=[
            pltpu.VMEM((_K, 128), jnp.float32),
        ],
        interpret=interpret,
    )(x, codebook)


def _gather(cn_pad, idx):
    info = plsc.get_sparse_core_info()
    nc, ns = info.num_cores, info.num_subcores
    nw = nc * ns
    b_per_w = _N // nw
    mesh = plsc.VectorSubcoreMesh(core_axis_name="c", subcore_axis_name="s")

    @functools.partial(
        pl.kernel,
        out_type=jax.ShapeDtypeStruct((_N, 128), jnp.float32),
        mesh=mesh,
        scratch_types=[
            pltpu.VMEM((b_per_w,), jnp.int32),
            pltpu.VMEM((b_per_w, 128), jnp.float32),
            pltpu.SemaphoreType.DMA,
        ],
    )
    def gather_k(table_hbm, idx_hbm, out_hbm, idx_v, rows_v, sem):
        wid = lax.axis_index("s") * nc + lax.axis_index("c")
        base = wid * b_per_w
        pltpu.sync_copy(idx_hbm.at[pl.ds(base, b_per_w)], idx_v)
        pltpu.async_copy(table_hbm.at[idx_v], rows_v, sem).wait()
        pltpu.sync_copy(rows_v, out_hbm.at[pl.ds(base, b_per_w)])

    return gather_k(cn_pad, idx)


def kernel(x, codebook):
    idx, commit, cn_pad = _vq_argmax(x, codebook)
    q_flat = _gather(cn_pad, idx.reshape(_N))
    q = jnp.transpose(q_flat.reshape(_B, _L, 128)[:, :, :_D], (0, 2, 1))
    return q, commit.reshape(())
